# baseline (speedup 1.0000x reference)
import functools
import math

import jax
import jax.numpy as jnp
from jax.experimental import pallas as pl
from jax.experimental.pallas import tpu as pltpu

NUM_EXPERTS = 4
NUM_GATES = 4
TOP = 2
EMB = 192
HID = 2 * EMB
B, H, W = 4, 56, 56
BN_EPS = 1e-5
P0 = H * W
P = 3200
T = 640
NT = P // T
_BN_SCALE = 1.0 / math.sqrt(1.0 + BN_EPS)


def _routing_body(x_ref, g_ref, comb_ref, loss_ref):
    x0 = jnp.sum(x_ref[...], axis=2) * (1.0 / P0)
    iota = jax.lax.broadcasted_iota(jnp.int32, (B, NUM_EXPERTS), 1)
    loss = jnp.float32(0.0)
    for g in range(NUM_GATES):
        logits = jnp.dot(x0, g_ref[g], preferred_element_type=jnp.float32)
        m = jnp.max(logits, axis=1, keepdims=True)
        ex = jnp.exp(logits - m)
        p = ex / jnp.sum(ex, axis=1, keepdims=True)
        usage = jnp.sum(p, axis=0)
        mu = jnp.sum(usage) * (1.0 / NUM_EXPERTS)
        var = jnp.sum((usage - mu) ** 2) * (1.0 / (NUM_EXPERTS - 1))
        loss = loss + var / (mu * mu + 1e-10)
        m1 = jnp.max(p, axis=1, keepdims=True)
        i1 = jnp.min(jnp.where(p == m1, iota, NUM_EXPERTS), axis=1,
                     keepdims=True)
        oh1 = iota == i1
        pm = jnp.where(oh1, -jnp.inf, p)
        m2 = jnp.max(pm, axis=1, keepdims=True)
        i2 = jnp.min(jnp.where(pm == m2, iota, NUM_EXPERTS), axis=1,
                     keepdims=True)
        oh2 = iota == i2
        e2 = jnp.exp(m2 - m1)
        w1 = 1.0 / (1.0 + e2)
        w2 = e2 / (1.0 + e2)
        comb_ref[g] = jnp.where(oh1, w1, 0.0) + jnp.where(oh2, w2, 0.0)
    loss_ref[0, 0] = loss


def _main_body(comb_ref, x_ref, w1_ref, b1_ref, w2_ref, b2_ref, bng_ref,
               bnb_ref, w3_ref, b3_ref, out_ref):
    b = pl.program_id(0)
    x = x_ref[0]
    acc = [None] * NUM_GATES
    for e in range(NUM_EXPERTS):
        h1 = jnp.dot(w1_ref[e], x, preferred_element_type=jnp.float32)
        h1 = h1 + b1_ref[:, e:e + 1]
        h2 = jnp.dot(w2_ref[e], h1, preferred_element_type=jnp.float32)
        h2 = h2 * (bng_ref[:, e:e + 1] * _BN_SCALE) + (
            b2_ref[:, e:e + 1] * (bng_ref[:, e:e + 1] * _BN_SCALE)
            + bnb_ref[:, e:e + 1])
        h2 = jnp.maximum(h2, 0.0)
        y = jnp.dot(w3_ref[e], h2, preferred_element_type=jnp.float32)
        y = y + b3_ref[:, e:e + 1]
        for g in range(NUM_GATES):
            term = comb_ref[g, b, e] * y
            acc[g] = term if acc[g] is None else acc[g] + term
    for g in range(NUM_GATES):
        out_ref[g, 0] = acc[g]


@jax.jit
def kernel(x, gate1, gate2, gate3, gate4, W1, b1, W2, b2, bn_g, bn_b, W3, b3):
    xp = jnp.pad(x.reshape(B, EMB, P0), ((0, 0), (0, 0), (0, P - P0)))
    gates = jnp.stack([gate1, gate2, gate3, gate4])

    comb, loss = pl.pallas_call(
        _routing_body,
        out_shape=(
            jax.ShapeDtypeStruct((NUM_GATES, B, NUM_EXPERTS), jnp.float32),
            jax.ShapeDtypeStruct((1, 1), jnp.float32),
        ),
        in_specs=[
            pl.BlockSpec((B, EMB, P), lambda: (0, 0, 0)),
            pl.BlockSpec((NUM_GATES, EMB, NUM_EXPERTS), lambda: (0, 0, 0)),
        ],
        out_specs=(
            pl.BlockSpec((NUM_GATES, B, NUM_EXPERTS), lambda: (0, 0, 0)),
            pl.BlockSpec(memory_space=pltpu.SMEM),
        ),
    )(xp, gates)

    ys = pl.pallas_call(
        _main_body,
        grid=(B, NT),
        out_shape=jax.ShapeDtypeStruct((NUM_GATES, B, EMB, P), jnp.float32),
        in_specs=[
            pl.BlockSpec(memory_space=pltpu.SMEM),
            pl.BlockSpec((1, EMB, T), lambda b, t: (b, 0, t)),
            pl.BlockSpec((NUM_EXPERTS, HID, EMB), lambda b, t: (0, 0, 0)),
            pl.BlockSpec((HID, NUM_EXPERTS), lambda b, t: (0, 0)),
            pl.BlockSpec((NUM_EXPERTS, HID, HID), lambda b, t: (0, 0, 0)),
            pl.BlockSpec((HID, NUM_EXPERTS), lambda b, t: (0, 0)),
            pl.BlockSpec((HID, NUM_EXPERTS), lambda b, t: (0, 0)),
            pl.BlockSpec((HID, NUM_EXPERTS), lambda b, t: (0, 0)),
            pl.BlockSpec((NUM_EXPERTS, EMB, HID), lambda b, t: (0, 0, 0)),
            pl.BlockSpec((EMB, NUM_EXPERTS), lambda b, t: (0, 0)),
        ],
        out_specs=pl.BlockSpec((NUM_GATES, 1, EMB, T),
                               lambda b, t: (0, b, 0, t)),
    )(comb, xp, W1, b1.T, W2, b2.T, bn_g.T, bn_b.T, W3, b3.T)

    y = ys[..., :P0].reshape(NUM_GATES, B, EMB, H, W)
    return (y[0], y[1], y[2], y[3], loss.reshape(()))


# speedup vs baseline: 3.0893x; 3.0893x over previous
import functools
import math

import jax
import jax.numpy as jnp
from jax.experimental import pallas as pl
from jax.experimental.pallas import tpu as pltpu

NUM_EXPERTS = 4
NUM_GATES = 4
TOP = 2
EMB = 192
HID = 2 * EMB
B, H, W = 4, 56, 56
BN_EPS = 1e-5
P0 = H * W
_BN_SCALE = 1.0 / math.sqrt(1.0 + BN_EPS)


def _routing_body(x_ref, g1_ref, g2_ref, g3_ref, g4_ref, comb_ref, loss_ref):
    x0 = jnp.sum(x_ref[...], axis=2) * (1.0 / P0)
    iota = jax.lax.broadcasted_iota(jnp.int32, (B, NUM_EXPERTS), 1)
    loss = jnp.float32(0.0)
    gate_refs = (g1_ref, g2_ref, g3_ref, g4_ref)
    for g in range(NUM_GATES):
        logits = jnp.dot(x0, gate_refs[g][...],
                         preferred_element_type=jnp.float32)
        m = jnp.max(logits, axis=1, keepdims=True)
        ex = jnp.exp(logits - m)
        p = ex / jnp.sum(ex, axis=1, keepdims=True)
        usage = jnp.sum(p, axis=0)
        mu = jnp.sum(usage) * (1.0 / NUM_EXPERTS)
        var = jnp.sum((usage - mu) ** 2) * (1.0 / (NUM_EXPERTS - 1))
        loss = loss + var / (mu * mu + 1e-10)
        m1 = jnp.max(p, axis=1, keepdims=True)
        i1 = jnp.min(jnp.where(p == m1, iota, NUM_EXPERTS), axis=1,
                     keepdims=True)
        oh1 = iota == i1
        pm = jnp.where(oh1, -jnp.inf, p)
        m2 = jnp.max(pm, axis=1, keepdims=True)
        i2 = jnp.min(jnp.where(pm == m2, iota, NUM_EXPERTS), axis=1,
                     keepdims=True)
        oh2 = iota == i2
        e2 = jnp.exp(m2 - m1)
        w1 = 1.0 / (1.0 + e2)
        w2 = e2 / (1.0 + e2)
        comb_ref[g] = jnp.where(oh1, w1, 0.0) + jnp.where(oh2, w2, 0.0)
    loss_ref[0, 0] = loss


def _main_body(comb_ref, x_ref, w1_ref, b1_ref, w2_ref, b2_ref, bng_ref,
               bnb_ref, w3_ref, b3_ref, *out_ref):
    b = pl.program_id(0)
    x = x_ref[0]
    for e in range(NUM_EXPERTS):
        h1 = jnp.dot(w1_ref[e], x, preferred_element_type=jnp.float32)
        h1 = h1 + b1_ref[:, e:e + 1]
        h2 = jnp.dot(w2_ref[e], h1, preferred_element_type=jnp.float32)
        h2 = h2 * (bng_ref[:, e:e + 1] * _BN_SCALE) + (
            b2_ref[:, e:e + 1] * (bng_ref[:, e:e + 1] * _BN_SCALE)
            + bnb_ref[:, e:e + 1])
        h2 = jnp.maximum(h2, 0.0)
        y = jnp.dot(w3_ref[e], h2, preferred_element_type=jnp.float32)
        y = y + b3_ref[:, e:e + 1]
        for g in range(NUM_GATES):
            term = comb_ref[g, b, e] * y
            if e == 0:
                out_ref[g][0] = term
            else:
                out_ref[g][0] += term


@jax.jit
def kernel(x, gate1, gate2, gate3, gate4, W1, b1, W2, b2, bn_g, bn_b, W3, b3):
    xr = x.reshape(B, EMB, P0)

    comb, loss = pl.pallas_call(
        _routing_body,
        out_shape=(
            jax.ShapeDtypeStruct((NUM_GATES, B, NUM_EXPERTS), jnp.float32),
            jax.ShapeDtypeStruct((1, 1), jnp.float32),
        ),
        in_specs=[
            pl.BlockSpec((B, EMB, P0), lambda: (0, 0, 0)),
            pl.BlockSpec((EMB, NUM_EXPERTS), lambda: (0, 0)),
            pl.BlockSpec((EMB, NUM_EXPERTS), lambda: (0, 0)),
            pl.BlockSpec((EMB, NUM_EXPERTS), lambda: (0, 0)),
            pl.BlockSpec((EMB, NUM_EXPERTS), lambda: (0, 0)),
        ],
        out_specs=(
            pl.BlockSpec((NUM_GATES, B, NUM_EXPERTS), lambda: (0, 0, 0)),
            pl.BlockSpec(memory_space=pltpu.SMEM),
        ),
    )(xr, gate1, gate2, gate3, gate4)

    out_sds = jax.ShapeDtypeStruct((B, EMB, P0), jnp.float32)
    out_spec = pl.BlockSpec((1, EMB, P0), lambda b: (b, 0, 0))
    ys = pl.pallas_call(
        _main_body,
        grid=(B,),
        out_shape=(out_sds,) * NUM_GATES,
        in_specs=[
            pl.BlockSpec(memory_space=pltpu.SMEM),
            pl.BlockSpec((1, EMB, P0), lambda b: (b, 0, 0)),
            pl.BlockSpec((NUM_EXPERTS, HID, EMB), lambda b: (0, 0, 0)),
            pl.BlockSpec((HID, NUM_EXPERTS), lambda b: (0, 0)),
            pl.BlockSpec((NUM_EXPERTS, HID, HID), lambda b: (0, 0, 0)),
            pl.BlockSpec((HID, NUM_EXPERTS), lambda b: (0, 0)),
            pl.BlockSpec((HID, NUM_EXPERTS), lambda b: (0, 0)),
            pl.BlockSpec((HID, NUM_EXPERTS), lambda b: (0, 0)),
            pl.BlockSpec((NUM_EXPERTS, EMB, HID), lambda b: (0, 0, 0)),
            pl.BlockSpec((EMB, NUM_EXPERTS), lambda b: (0, 0)),
        ],
        out_specs=(out_spec,) * NUM_GATES,
    )(comb, xr, W1, b1.T, W2, b2.T, bn_g.T, bn_b.T, W3, b3.T)

    return tuple(yg.reshape(B, EMB, H, W) for yg in ys) + (loss.reshape(()),)


# speedup vs baseline: 4.9311x; 1.5962x over previous
import math

import jax
import jax.numpy as jnp
from jax.experimental import pallas as pl
from jax.experimental.pallas import tpu as pltpu

NUM_EXPERTS = 4
NUM_GATES = 4
TOP = 2
EMB = 192
HID = 2 * EMB
B, H, W = 4, 56, 56
BN_EPS = 1e-5
P0 = H * W
TP = 784
NT = P0 // TP
_BN_SCALE = 1.0 / math.sqrt(1.0 + BN_EPS)


def _body(xf_ref, x_ref, g1_ref, g2_ref, g3_ref, g4_ref, w1_ref, b1_ref,
          w2_ref, b2_ref, bng_ref, bnb_ref, w3_ref, b3_ref,
          o1_ref, o2_ref, o3_ref, o4_ref, loss_ref, comb_s, usage_s):
    b = pl.program_id(0)
    t = pl.program_id(1)
    out_ref = (o1_ref, o2_ref, o3_ref, o4_ref)
    gate_refs = (g1_ref, g2_ref, g3_ref, g4_ref)

    @pl.when(t == 0)
    def _routing():
        x0 = jnp.sum(xf_ref[0], axis=0, keepdims=True) * (1.0 / P0)
        iota = jax.lax.broadcasted_iota(jnp.int32, (1, NUM_EXPERTS), 1)
        for g in range(NUM_GATES):
            logits = jnp.dot(x0, gate_refs[g][...],
                             preferred_element_type=jnp.float32)
            m = jnp.max(logits, axis=1, keepdims=True)
            ex = jnp.exp(logits - m)
            p = ex / jnp.sum(ex, axis=1, keepdims=True)

            @pl.when(b == 0)
            def _(g=g, p=p):
                usage_s[g] = p

            @pl.when(b != 0)
            def _(g=g, p=p):
                usage_s[g] += p

            m1 = jnp.max(p, axis=1, keepdims=True)
            i1 = jnp.min(jnp.where(p == m1, iota, NUM_EXPERTS), axis=1,
                         keepdims=True)
            oh1 = iota == i1
            pm = jnp.where(oh1, -jnp.inf, p)
            m2 = jnp.max(pm, axis=1, keepdims=True)
            i2 = jnp.min(jnp.where(pm == m2, iota, NUM_EXPERTS), axis=1,
                         keepdims=True)
            oh2 = iota == i2
            e2 = jnp.exp(m2 - m1)
            w1 = 1.0 / (1.0 + e2)
            w2 = e2 / (1.0 + e2)
            comb_s[g, pl.ds(b, 1), :] = (jnp.where(oh1, w1, 0.0)
                                         + jnp.where(oh2, w2, 0.0))

    @pl.when(t > 0)
    def _experts():
        x = x_ref[0]
        dims = (((1,), (1,)), ((), ()))
        for e in range(NUM_EXPERTS):
            h1 = jax.lax.dot_general(x, w1_ref[e], dims,
                                     preferred_element_type=jnp.float32)
            h1 = h1 + b1_ref[e:e + 1, :]
            h2 = jax.lax.dot_general(h1, w2_ref[e], dims,
                                     preferred_element_type=jnp.float32)
            scale = bng_ref[e:e + 1, :] * _BN_SCALE
            h2 = h2 * scale + (b2_ref[e:e + 1, :] * scale
                               + bnb_ref[e:e + 1, :])
            h2 = jnp.maximum(h2, 0.0)
            y = jax.lax.dot_general(h2, w3_ref[e], dims,
                                    preferred_element_type=jnp.float32)
            y = y + b3_ref[e:e + 1, :]
            for g in range(NUM_GATES):
                term = y * comb_s[g, pl.ds(b, 1), e:e + 1]
                if e == 0:
                    out_ref[g][0] = term
                else:
                    out_ref[g][0] += term

    @pl.when(jnp.logical_and(b == B - 1, t == NT))
    def _loss():
        total = jnp.float32(0.0)
        for g in range(NUM_GATES):
            u = usage_s[g]
            mu = jnp.sum(u) * (1.0 / NUM_EXPERTS)
            var = jnp.sum((u - mu) ** 2) * (1.0 / (NUM_EXPERTS - 1))
            total = total + var / (mu * mu + 1e-10)
        loss_ref[0, 0] = total


@jax.jit
def kernel(x, gate1, gate2, gate3, gate4, W1, b1, W2, b2, bn_g, bn_b, W3, b3):
    xv = x.transpose(0, 2, 3, 1).reshape(B, P0, EMB)

    out_sds = jax.ShapeDtypeStruct((B, P0, EMB), jnp.float32)
    out_spec = pl.BlockSpec((1, TP, EMB),
                            lambda b, t: (b, jnp.maximum(t - 1, 0), 0))
    gate_spec = pl.BlockSpec((EMB, NUM_EXPERTS), lambda b, t: (0, 0))
    vec_spec = pl.BlockSpec((NUM_EXPERTS, HID), lambda b, t: (0, 0))
    *ys, loss = pl.pallas_call(
        _body,
        grid=(B, NT + 1),
        out_shape=(out_sds,) * NUM_GATES
        + (jax.ShapeDtypeStruct((1, 1), jnp.float32),),
        in_specs=[
            pl.BlockSpec((1, P0, EMB), lambda b, t: (b, 0, 0)),
            pl.BlockSpec((1, TP, EMB),
                         lambda b, t: (b, jnp.maximum(t - 1, 0), 0)),
            gate_spec, gate_spec, gate_spec, gate_spec,
            pl.BlockSpec((NUM_EXPERTS, HID, EMB), lambda b, t: (0, 0, 0)),
            vec_spec,
            pl.BlockSpec((NUM_EXPERTS, HID, HID), lambda b, t: (0, 0, 0)),
            vec_spec, vec_spec, vec_spec,
            pl.BlockSpec((NUM_EXPERTS, EMB, HID), lambda b, t: (0, 0, 0)),
            pl.BlockSpec((NUM_EXPERTS, EMB), lambda b, t: (0, 0)),
        ],
        out_specs=(out_spec,) * NUM_GATES
        + (pl.BlockSpec(memory_space=pltpu.SMEM),),
        scratch_shapes=[
            pltpu.VMEM((NUM_GATES, B, NUM_EXPERTS), jnp.float32),
            pltpu.VMEM((NUM_GATES, 1, NUM_EXPERTS), jnp.float32),
        ],
    )(xv, xv, gate1, gate2, gate3, gate4, W1, b1, W2, b2, bn_g, bn_b, W3, b3)

    out = tuple(
        yg.reshape(B, H, W, EMB).transpose(0, 3, 1, 2) for yg in ys
    )
    return out + (loss.reshape(()),)


# speedup vs baseline: 5.3734x; 1.0897x over previous
import math

import jax
import jax.numpy as jnp
from jax.experimental import pallas as pl
from jax.experimental.pallas import tpu as pltpu

NUM_EXPERTS = 4
NUM_GATES = 4
TOP = 2
EMB = 192
HID = 2 * EMB
B, H, W = 4, 56, 56
BN_EPS = 1e-5
P0 = H * W
TP = 784
NT = P0 // TP
_BN_SCALE = 1.0 / math.sqrt(1.0 + BN_EPS)


def _routing_body(x_ref, g1_ref, g2_ref, g3_ref, g4_ref, comb_ref, loss_ref):
    x0 = jnp.sum(x_ref[...], axis=1) * (1.0 / P0)
    iota = jax.lax.broadcasted_iota(jnp.int32, (B, NUM_EXPERTS), 1)
    loss = jnp.float32(0.0)
    gate_refs = (g1_ref, g2_ref, g3_ref, g4_ref)
    for g in range(NUM_GATES):
        logits = jnp.dot(x0, gate_refs[g][...],
                         preferred_element_type=jnp.float32)
        m = jnp.max(logits, axis=1, keepdims=True)
        ex = jnp.exp(logits - m)
        p = ex / jnp.sum(ex, axis=1, keepdims=True)
        usage = jnp.sum(p, axis=0)
        mu = jnp.sum(usage) * (1.0 / NUM_EXPERTS)
        var = jnp.sum((usage - mu) ** 2) * (1.0 / (NUM_EXPERTS - 1))
        loss = loss + var / (mu * mu + 1e-10)
        m1 = jnp.max(p, axis=1, keepdims=True)
        i1 = jnp.min(jnp.where(p == m1, iota, NUM_EXPERTS), axis=1,
                     keepdims=True)
        oh1 = iota == i1
        pm = jnp.where(oh1, -jnp.inf, p)
        m2 = jnp.max(pm, axis=1, keepdims=True)
        i2 = jnp.min(jnp.where(pm == m2, iota, NUM_EXPERTS), axis=1,
                     keepdims=True)
        oh2 = iota == i2
        e2 = jnp.exp(m2 - m1)
        w1 = 1.0 / (1.0 + e2)
        w2 = e2 / (1.0 + e2)
        comb_ref[g] = jnp.where(oh1, w1, 0.0) + jnp.where(oh2, w2, 0.0)
    loss_ref[0, 0] = loss


def _main_body(comb_ref, x_ref, w1_ref, b1_ref, w2_ref, b2_ref, bng_ref,
               bnb_ref, w3_ref, b3_ref, *out_ref):
    b = pl.program_id(0)
    x = x_ref[0]
    dims = (((1,), (1,)), ((), ()))
    for e in range(NUM_EXPERTS):
        h1 = jax.lax.dot_general(x, w1_ref[e], dims,
                                 preferred_element_type=jnp.float32)
        h1 = h1 + b1_ref[e:e + 1, :]
        h2 = jax.lax.dot_general(h1, w2_ref[e], dims,
                                 preferred_element_type=jnp.float32)
        scale = bng_ref[e:e + 1, :] * _BN_SCALE
        h2 = h2 * scale + (b2_ref[e:e + 1, :] * scale + bnb_ref[e:e + 1, :])
        h2 = jnp.maximum(h2, 0.0)
        y = jax.lax.dot_general(h2, w3_ref[e], dims,
                                preferred_element_type=jnp.float32)
        y = y + b3_ref[e:e + 1, :]
        for g in range(NUM_GATES):
            term = comb_ref[g, b, e] * y
            if e == 0:
                out_ref[g][0] = term
            else:
                out_ref[g][0] += term


@jax.jit
def kernel(x, gate1, gate2, gate3, gate4, W1, b1, W2, b2, bn_g, bn_b, W3, b3):
    xv = x.transpose(0, 2, 3, 1).reshape(B, P0, EMB)

    comb, loss = pl.pallas_call(
        _routing_body,
        out_shape=(
            jax.ShapeDtypeStruct((NUM_GATES, B, NUM_EXPERTS), jnp.float32),
            jax.ShapeDtypeStruct((1, 1), jnp.float32),
        ),
        in_specs=[
            pl.BlockSpec((B, P0, EMB), lambda: (0, 0, 0)),
            pl.BlockSpec((EMB, NUM_EXPERTS), lambda: (0, 0)),
            pl.BlockSpec((EMB, NUM_EXPERTS), lambda: (0, 0)),
            pl.BlockSpec((EMB, NUM_EXPERTS), lambda: (0, 0)),
            pl.BlockSpec((EMB, NUM_EXPERTS), lambda: (0, 0)),
        ],
        out_specs=(
            pl.BlockSpec((NUM_GATES, B, NUM_EXPERTS), lambda: (0, 0, 0)),
            pl.BlockSpec(memory_space=pltpu.SMEM),
        ),
    )(xv, gate1, gate2, gate3, gate4)

    out_sds = jax.ShapeDtypeStruct((B, P0, EMB), jnp.float32)
    out_spec = pl.BlockSpec((1, TP, EMB), lambda b, t: (b, t, 0))
    ys = pl.pallas_call(
        _main_body,
        grid=(B, NT),
        out_shape=(out_sds,) * NUM_GATES,
        in_specs=[
            pl.BlockSpec(memory_space=pltpu.SMEM),
            pl.BlockSpec((1, TP, EMB), lambda b, t: (b, t, 0)),
            pl.BlockSpec((NUM_EXPERTS, HID, EMB), lambda b, t: (0, 0, 0)),
            pl.BlockSpec((NUM_EXPERTS, HID), lambda b, t: (0, 0)),
            pl.BlockSpec((NUM_EXPERTS, HID, HID), lambda b, t: (0, 0, 0)),
            pl.BlockSpec((NUM_EXPERTS, HID), lambda b, t: (0, 0)),
            pl.BlockSpec((NUM_EXPERTS, HID), lambda b, t: (0, 0)),
            pl.BlockSpec((NUM_EXPERTS, HID), lambda b, t: (0, 0)),
            pl.BlockSpec((NUM_EXPERTS, EMB, HID), lambda b, t: (0, 0, 0)),
            pl.BlockSpec((NUM_EXPERTS, EMB), lambda b, t: (0, 0)),
        ],
        out_specs=(out_spec,) * NUM_GATES,
    )(comb, xv, W1, b1, W2, b2, bn_g, bn_b, W3, b3)

    out = tuple(
        yg.reshape(B, H, W, EMB).transpose(0, 3, 1, 2) for yg in ys
    )
    return out + (loss.reshape(()),)
